# baseline (speedup 1.0000x reference)
import functools

import jax
import jax.numpy as jnp
import numpy as np
from jax.experimental import pallas as pl
from jax.experimental.pallas import tpu as pltpu

_K_NEIGHBORS = 16
_NEG = -1e30
_BN = 8


def _hyper_kernel(x_ref, wq_ref, bq_ref, kp_ref, out_ref, *, t_dim):
    inter, m_dim = kp_ref.shape
    bn, v_dim, _, c_dim = x_ref.shape

    hs = []
    for b in range(bn):
        x2 = x_ref[b].reshape(v_dim * t_dim, c_dim)
        p = jax.lax.dot_general(
            x2, wq_ref[...], (((1,), (1,)), ((), ())),
            preferred_element_type=jnp.float32)
        a = p.reshape(v_dim, t_dim, inter)
        half = t_dim
        while half > 1:
            half //= 2
            a = a[:, :half, :] + a[:, half:, :]
        q = a[:, 0, :]
        q = q * (1.0 / t_dim) + bq_ref[...]
        norm = jnp.sqrt(jnp.sum(q * q, axis=1, keepdims=True))
        qn = q / jnp.maximum(norm, 1e-12)
        h_b = jnp.dot(qn, kp_ref[...],
                      preferred_element_type=jnp.float32) * (inter ** -0.5)
        hs.append(h_b[None])
    h = jnp.concatenate(hs, axis=0)

    iota = jax.lax.broadcasted_iota(
        jnp.int32, (bn, v_dim, m_dim), 2).astype(jnp.float32)
    work = h
    rmax = None
    for _ in range(_K_NEIGHBORS):
        mval = jnp.max(work, axis=2, keepdims=True)
        if rmax is None:
            rmax = mval
        ismax = work == mval
        idx = jnp.min(jnp.where(ismax, iota, float(m_dim)), axis=2,
                      keepdims=True)
        work = jnp.where(iota == idx, _NEG, work)
    mask = work == _NEG
    ex = jnp.where(mask, jnp.exp(h - rmax), 0.0)
    denom = jnp.sum(ex, axis=2, keepdims=True)
    out_ref[...] = ex / denom


def kernel(x, W_q, b_q, key_prototypes):
    N, C, T, V = x.shape
    inter, M = key_prototypes.shape
    xt = jnp.transpose(x, (0, 3, 2, 1))
    bq2 = b_q.reshape(1, inter)

    grid = (N // _BN,)
    return pl.pallas_call(
        functools.partial(_hyper_kernel, t_dim=T),
        grid=grid,
        in_specs=[
            pl.BlockSpec((_BN, V, T, C), lambda i: (i, 0, 0, 0)),
            pl.BlockSpec((inter, C), lambda i: (0, 0)),
            pl.BlockSpec((1, inter), lambda i: (0, 0)),
            pl.BlockSpec((inter, M), lambda i: (0, 0)),
        ],
        out_specs=pl.BlockSpec((_BN, V, M), lambda i: (i, 0, 0)),
        out_shape=jax.ShapeDtypeStruct((N, V, M), jnp.float32),
        compiler_params=pltpu.CompilerParams(
            dimension_semantics=("arbitrary",),
        ),
    )(xt, W_q, bq2, key_prototypes)


# speedup vs baseline: 1.4583x; 1.4583x over previous
import functools

import jax
import jax.numpy as jnp
import numpy as np
from jax.experimental import pallas as pl
from jax.experimental.pallas import tpu as pltpu

_K_NEIGHBORS = 16
_NEG = -1e30
_BN = 8


def _hyper_kernel(x_ref, wq_ref, bq_ref, kp_ref, out_ref, *, t_dim):
    inter, m_dim = kp_ref.shape
    bn, v_dim, _, c_dim = x_ref.shape

    hs = []
    for b in range(bn):
        x2 = x_ref[b].reshape(v_dim * t_dim, c_dim)
        p = jax.lax.dot_general(
            x2, wq_ref[...], (((1,), (1,)), ((), ())),
            preferred_element_type=jnp.float32)
        q = jnp.sum(p.reshape(v_dim, t_dim, inter), axis=1)
        q = q * (1.0 / t_dim) + bq_ref[...]
        norm = jnp.sqrt(jnp.sum(q * q, axis=1, keepdims=True))
        qn = q / jnp.maximum(norm, 1e-12)
        h_b = jnp.dot(qn, kp_ref[...],
                      preferred_element_type=jnp.float32) * (inter ** -0.5)
        hs.append(h_b[None])
    h = jnp.concatenate(hs, axis=0)

    iota = jax.lax.broadcasted_iota(
        jnp.int32, (bn, v_dim, m_dim), 2).astype(jnp.float32)
    work = h
    rmax = None
    for _ in range(_K_NEIGHBORS):
        mval = jnp.max(work, axis=2, keepdims=True)
        if rmax is None:
            rmax = mval
        ismax = work == mval
        idx = jnp.min(jnp.where(ismax, iota, float(m_dim)), axis=2,
                      keepdims=True)
        work = jnp.where(iota == idx, _NEG, work)
    mask = work == _NEG
    ex = jnp.where(mask, jnp.exp(h - rmax), 0.0)
    denom = jnp.sum(ex, axis=2, keepdims=True)
    out_ref[...] = ex / denom


def kernel(x, W_q, b_q, key_prototypes):
    N, C, T, V = x.shape
    inter, M = key_prototypes.shape
    xt = jnp.transpose(x, (0, 3, 2, 1))
    bq2 = b_q.reshape(1, inter)

    grid = (N // _BN,)
    return pl.pallas_call(
        functools.partial(_hyper_kernel, t_dim=T),
        grid=grid,
        in_specs=[
            pl.BlockSpec((_BN, V, T, C), lambda i: (i, 0, 0, 0)),
            pl.BlockSpec((inter, C), lambda i: (0, 0)),
            pl.BlockSpec((1, inter), lambda i: (0, 0)),
            pl.BlockSpec((inter, M), lambda i: (0, 0)),
        ],
        out_specs=pl.BlockSpec((_BN, V, M), lambda i: (i, 0, 0)),
        out_shape=jax.ShapeDtypeStruct((N, V, M), jnp.float32),
        compiler_params=pltpu.CompilerParams(
            dimension_semantics=("arbitrary",),
        ),
    )(xt, W_q, bq2, key_prototypes)


# speedup vs baseline: 1.6210x; 1.1116x over previous
import functools

import jax
import jax.numpy as jnp
import numpy as np
from jax.experimental import pallas as pl
from jax.experimental.pallas import tpu as pltpu

_K_NEIGHBORS = 16
_NEG = -1e30
_BN = 16


def _hyper_kernel(x_ref, wq_ref, bq_ref, kp_ref, out_ref, *, t_dim):
    inter, m_dim = kp_ref.shape
    bn, v_dim, _, c_dim = x_ref.shape

    hs = []
    for b in range(bn):
        x2 = x_ref[b].reshape(v_dim * t_dim, c_dim)
        p = jax.lax.dot_general(
            x2, wq_ref[...], (((1,), (1,)), ((), ())),
            preferred_element_type=jnp.float32)
        q = jnp.sum(p.reshape(v_dim, t_dim, inter), axis=1)
        q = q * (1.0 / t_dim) + bq_ref[...]
        norm = jnp.sqrt(jnp.sum(q * q, axis=1, keepdims=True))
        qn = q / jnp.maximum(norm, 1e-12)
        h_b = jnp.dot(qn, kp_ref[...],
                      preferred_element_type=jnp.float32) * (inter ** -0.5)
        hs.append(h_b[None])
    h = jnp.concatenate(hs, axis=0)

    iota = jax.lax.broadcasted_iota(
        jnp.int32, (bn, v_dim, m_dim), 2).astype(jnp.float32)
    work = h
    rmax = None
    for _ in range(_K_NEIGHBORS):
        mval = jnp.max(work, axis=2, keepdims=True)
        if rmax is None:
            rmax = mval
        ismax = work == mval
        idx = jnp.min(jnp.where(ismax, iota, float(m_dim)), axis=2,
                      keepdims=True)
        work = jnp.where(iota == idx, _NEG, work)
    mask = work == _NEG
    ex = jnp.where(mask, jnp.exp(h - rmax), 0.0)
    denom = jnp.sum(ex, axis=2, keepdims=True)
    out_ref[...] = ex / denom


def kernel(x, W_q, b_q, key_prototypes):
    N, C, T, V = x.shape
    inter, M = key_prototypes.shape
    xt = jnp.transpose(x, (0, 3, 2, 1))
    bq2 = b_q.reshape(1, inter)

    grid = (N // _BN,)
    return pl.pallas_call(
        functools.partial(_hyper_kernel, t_dim=T),
        grid=grid,
        in_specs=[
            pl.BlockSpec((_BN, V, T, C), lambda i: (i, 0, 0, 0)),
            pl.BlockSpec((inter, C), lambda i: (0, 0)),
            pl.BlockSpec((1, inter), lambda i: (0, 0)),
            pl.BlockSpec((inter, M), lambda i: (0, 0)),
        ],
        out_specs=pl.BlockSpec((_BN, V, M), lambda i: (i, 0, 0)),
        out_shape=jax.ShapeDtypeStruct((N, V, M), jnp.float32),
        compiler_params=pltpu.CompilerParams(
            dimension_semantics=("arbitrary",),
        ),
    )(xt, W_q, bq2, key_prototypes)


# speedup vs baseline: 1.8643x; 1.1501x over previous
import functools

import jax
import jax.numpy as jnp
import numpy as np
from jax.experimental import pallas as pl
from jax.experimental.pallas import tpu as pltpu

_K_NEIGHBORS = 16
_NEG = -1e30
_BN = 16


def _hyper_kernel(x_ref, wq_ref, bq_ref, kp_ref, out_ref, *, t_dim):
    inter, m_dim = kp_ref.shape
    bn, v_dim, _, c_dim = x_ref.shape

    hs = []
    for b in range(bn):
        x2 = x_ref[b].reshape(v_dim * t_dim, c_dim)
        p = jax.lax.dot_general(
            x2, wq_ref[...], (((1,), (1,)), ((), ())),
            preferred_element_type=jnp.float32)
        q = jnp.sum(p.reshape(v_dim, t_dim, inter), axis=1)
        q = q * (1.0 / t_dim) + bq_ref[...]
        norm = jnp.sqrt(jnp.sum(q * q, axis=1, keepdims=True))
        qn = q / jnp.maximum(norm, 1e-12)
        h_b = jnp.dot(qn, kp_ref[...],
                      preferred_element_type=jnp.float32) * (inter ** -0.5)
        hs.append(h_b[None])
    h = jnp.concatenate(hs, axis=0)

    work = h
    rmax = None
    for _ in range(_K_NEIGHBORS):
        mval = jnp.max(work, axis=2, keepdims=True)
        if rmax is None:
            rmax = mval
        work = jnp.where(work == mval, _NEG, work)
    mask = work == _NEG
    ex = jnp.where(mask, jnp.exp(h - rmax), 0.0)
    denom = jnp.sum(ex, axis=2, keepdims=True)
    out_ref[...] = ex / denom


def kernel(x, W_q, b_q, key_prototypes):
    N, C, T, V = x.shape
    inter, M = key_prototypes.shape
    xt = jnp.transpose(x, (0, 3, 2, 1))
    bq2 = b_q.reshape(1, inter)

    grid = (N // _BN,)
    return pl.pallas_call(
        functools.partial(_hyper_kernel, t_dim=T),
        grid=grid,
        in_specs=[
            pl.BlockSpec((_BN, V, T, C), lambda i: (i, 0, 0, 0)),
            pl.BlockSpec((inter, C), lambda i: (0, 0)),
            pl.BlockSpec((1, inter), lambda i: (0, 0)),
            pl.BlockSpec((inter, M), lambda i: (0, 0)),
        ],
        out_specs=pl.BlockSpec((_BN, V, M), lambda i: (i, 0, 0)),
        out_shape=jax.ShapeDtypeStruct((N, V, M), jnp.float32),
        compiler_params=pltpu.CompilerParams(
            dimension_semantics=("arbitrary",),
        ),
    )(xt, W_q, bq2, key_prototypes)
